# Initial kernel scaffold; baseline (speedup 1.0000x reference)
import jax
import jax.numpy as jnp
from jax.experimental import pallas as pl


def kernel(x_nchw, w1, b1, w2, b2):
    raise NotImplementedError("write your pallas kernel here")


# speedup vs baseline: 1.2648x; 1.2648x over previous
import jax
import jax.numpy as jnp
from jax.experimental import pallas as pl
from jax.experimental.pallas import tpu as pltpu

_VMEM_BUDGET = int(64 * 1024 * 1024 * 0.7)


def _make_se_kernel(B, inv_hw):
    def se_kernel(x_ref, w1_ref, b1_ref, w2_ref, b2_ref, o_ref):
        C = x_ref.shape[1]
        HW = x_ref.shape[2]

        x_flat = x_ref[...].reshape(B * C, HW)
        ones = jnp.ones((HW, 1), dtype=x_flat.dtype)
        pooled_col = jnp.dot(x_flat, ones, preferred_element_type=jnp.float32)

        pooled = jnp.concatenate(
            [pooled_col[b * C:(b + 1) * C, :] for b in range(B)], axis=1)
        pooled = pooled * inv_hw

        h = jnp.dot(w1_ref[...], pooled,
                    preferred_element_type=jnp.float32) + b1_ref[...]
        h = h * jax.nn.sigmoid(h)
        g = jnp.dot(w2_ref[...], h,
                    preferred_element_type=jnp.float32) + b2_ref[...]
        g = jax.nn.sigmoid(g).astype(o_ref.dtype)

        for b in range(B):
            o_ref[b] = x_ref[b] * g[:, b:b + 1]

    return se_kernel


def _pick_batch(N, slab_bytes, w_bytes):
    for B in (8, 4, 2, 1):
        if N % B:
            continue
        if 4 * B * slab_bytes + 2 * w_bytes + (2 << 20) <= _VMEM_BUDGET:
            return B
    return 1


def kernel(x_nchw, w1, b1, w2, b2):
    N, C, H, W = x_nchw.shape
    Cse = w1.shape[0]
    HW = H * W
    itemsize = jnp.dtype(x_nchw.dtype).itemsize

    x3 = x_nchw.reshape(N, C, HW)
    w1f = w1.astype(jnp.float32)
    w2f = w2.astype(jnp.float32)
    b1c = b1.reshape(Cse, 1).astype(jnp.float32)
    b2c = b2.reshape(C, 1).astype(jnp.float32)

    slab = C * HW * itemsize
    w_bytes = (Cse * C + C * Cse + Cse + C) * 4
    B = _pick_batch(N, slab, w_bytes)

    out = pl.pallas_call(
        _make_se_kernel(B, 1.0 / float(HW)),
        out_shape=jax.ShapeDtypeStruct((N, C, HW), x_nchw.dtype),
        grid=(N // B,),
        in_specs=[
            pl.BlockSpec((B, C, HW), lambda n: (n, 0, 0)),
            pl.BlockSpec((Cse, C), lambda n: (0, 0)),
            pl.BlockSpec((Cse, 1), lambda n: (0, 0)),
            pl.BlockSpec((C, Cse), lambda n: (0, 0)),
            pl.BlockSpec((C, 1), lambda n: (0, 0)),
        ],
        out_specs=pl.BlockSpec((B, C, HW), lambda n: (n, 0, 0)),
        compiler_params=pltpu.CompilerParams(
            dimension_semantics=("parallel",),
            vmem_limit_bytes=_VMEM_BUDGET),
    )(x3, w1f, b1c, w2f, b2c)
    return out.reshape(N, C, H, W)
